# baseline (speedup 1.0000x reference)
import functools

import jax
import jax.numpy as jnp
from jax import lax
from jax.experimental import pallas as pl
from jax.experimental.pallas import tpu as pltpu
from jax.experimental.pallas import tpu_sc as plsc

B = 16384
D = 64
NUM_CORES = 2
NUM_SUBCORES = 16
NW = NUM_CORES * NUM_SUBCORES
BPW = B // NW
CHUNK = 128
NCHUNK = BPW // CHUNK


def _sc_gather(user_table, item_table, uidx2d, iidx2d):
    mesh = plsc.VectorSubcoreMesh(
        core_axis_name="c", subcore_axis_name="s",
        num_cores=NUM_CORES, num_subcores=NUM_SUBCORES,
    )

    @functools.partial(
        pl.kernel,
        out_type=(
            jax.ShapeDtypeStruct((B, D), jnp.float32),
            jax.ShapeDtypeStruct((B, D), jnp.float32),
        ),
        mesh=mesh,
        compiler_params=pltpu.CompilerParams(use_tc_tiling_on_sc=False),
        scratch_types=[
            pltpu.VMEM((NCHUNK, CHUNK), jnp.int32),
            pltpu.VMEM((NCHUNK, CHUNK), jnp.int32),
            pltpu.VMEM((BPW, D), jnp.float32),
            pltpu.VMEM((BPW, D), jnp.float32),
            pltpu.SemaphoreType.DMA,
        ],
    )
    def gather_kernel(u_tab, i_tab, u_idx, i_idx, out_u, out_i,
                      uidx_v, iidx_v, urows_v, irows_v, sem):
        wid = lax.axis_index("s") * NUM_CORES + lax.axis_index("c")
        base = wid * BPW
        crow = wid * NCHUNK
        pltpu.sync_copy(u_idx.at[pl.ds(crow, NCHUNK)], uidx_v)
        pltpu.sync_copy(i_idx.at[pl.ds(crow, NCHUNK)], iidx_v)
        copies = []
        for j in range(NCHUNK):
            copies.append(pltpu.async_copy(
                u_tab.at[uidx_v.at[j]],
                urows_v.at[pl.ds(j * CHUNK, CHUNK)], sem))
            copies.append(pltpu.async_copy(
                i_tab.at[iidx_v.at[j]],
                irows_v.at[pl.ds(j * CHUNK, CHUNK)], sem))
        for c in copies:
            c.wait()
        pltpu.sync_copy(urows_v, out_u.at[pl.ds(base, BPW)])
        pltpu.sync_copy(irows_v, out_i.at[pl.ds(base, BPW)])

    return gather_kernel(user_table, item_table, uidx2d, iidx2d)


def _combine_body(u_ref, v_ref, o_ref):
    u = u_ref[...]
    v = v_ref[...]
    mu = jnp.mean(u, axis=0, keepdims=True)
    mv = jnp.mean(v, axis=0, keepdims=True)
    fu = u + 7.0 * mu
    fv = v + 7.0 * mv
    inter = jnp.sum(fu * fv, axis=1, keepdims=True) * (1.0 / 16.0)
    o_ref[...] = jax.nn.sigmoid(inter)


def kernel(user_indices, item_indices, user_table, item_table):
    uidx2d = user_indices.reshape(B // CHUNK, CHUNK)
    iidx2d = item_indices.reshape(B // CHUNK, CHUNK)
    u_rows, i_rows = _sc_gather(user_table, item_table, uidx2d, iidx2d)
    out = pl.pallas_call(
        _combine_body,
        out_shape=jax.ShapeDtypeStruct((B, 1), jnp.float32),
    )(u_rows, i_rows)
    return out.reshape(B)
